# baseline (speedup 1.0000x reference)
import functools

import jax
import jax.numpy as jnp
from jax import lax
from jax.experimental import pallas as pl
from jax.experimental.pallas import tpu as pltpu
from jax.experimental.pallas import tpu_sc as plsc


@functools.cache
def _make_gather(Bt: int, S: int, D: int):
    info = plsc.get_sparse_core_info()
    NC, NS = info.num_cores, info.num_subcores
    NW = NC * NS
    assert Bt % NW == 0
    RW = Bt // NW

    NB = 8
    K = 6
    assert RW % NB == 0 and K < NB
    mesh = plsc.VectorSubcoreMesh(core_axis_name="c", subcore_axis_name="s")

    @functools.partial(
        pl.kernel,
        mesh=mesh,
        out_type=jax.ShapeDtypeStruct((Bt, S, D), jnp.float32),
        compiler_params=pltpu.CompilerParams(use_tc_tiling_on_sc=False),
        scratch_types=[
            pltpu.VMEM((RW, S), jnp.int32),
            pltpu.VMEM((NB, S, D), jnp.float32),
            pltpu.SemaphoreType.DMA((NB,)),
            pltpu.SemaphoreType.DMA((NB,)),
        ],
    )
    def gather_kernel(idx_hbm, table_hbm, out_hbm, idx_v, rows_v, g_sems, w_sems):
        wid = lax.axis_index("s") * NC + lax.axis_index("c")
        base = wid * RW
        pltpu.sync_copy(idx_hbm.at[pl.ds(base, RW)], idx_v)

        def start_gather(j, slot):
            pltpu.async_copy(
                table_hbm.at[idx_v.at[j]], rows_v.at[slot], g_sems.at[slot]
            )

        def wait_gather(slot):
            pltpu.make_async_copy(
                table_hbm.at[idx_v.at[0]], rows_v.at[slot], g_sems.at[slot]
            ).wait()

        def wait_write(slot):
            pltpu.make_async_copy(
                rows_v.at[slot], out_hbm.at[base], w_sems.at[slot]
            ).wait()

        for c in range(K):
            start_gather(c, c)

        def outer(o, carry):
            for b in range(NB):
                j = o * NB + b
                s = (b + K) % NB
                @pl.when(j + K < RW)
                def _():
                    @pl.when(j + K >= NB)
                    def _():
                        wait_write(s)
                    start_gather(j + K, s)
                wait_gather(b)
                pltpu.async_copy(
                    rows_v.at[b], out_hbm.at[base + j], w_sems.at[b]
                )
            return carry

        lax.fori_loop(0, RW // NB, outer, 0)
        for b in range(NB):
            wait_write(b)

    return gather_kernel


def kernel(input_, weight):
    Bt, S = input_.shape
    D = weight.shape[1]
    return _make_gather(Bt, S, D)(input_, weight)


# speedup vs baseline: 1.0610x; 1.0610x over previous
import functools

import jax
import jax.numpy as jnp
from jax import lax
from jax.experimental import pallas as pl
from jax.experimental.pallas import tpu as pltpu
from jax.experimental.pallas import tpu_sc as plsc


@functools.cache
def _make_gather(Bt: int, S: int, D: int):
    info = plsc.get_sparse_core_info()
    NC, NS = info.num_cores, info.num_subcores
    NW = NC * NS
    assert Bt % NW == 0
    RW = Bt // NW

    NB = 8
    K = 6
    assert RW % NB == 0 and K < NB
    mesh = plsc.VectorSubcoreMesh(core_axis_name="c", subcore_axis_name="s")

    @functools.partial(
        pl.kernel,
        mesh=mesh,
        out_type=jax.ShapeDtypeStruct((Bt, S, D), jnp.float32),
        compiler_params=pltpu.CompilerParams(use_tc_tiling_on_sc=False),
        scratch_types=[
            pltpu.VMEM((RW, S), jnp.int32),
            pltpu.VMEM((NB, S, 2 * D), jnp.float32),
            pltpu.SemaphoreType.DMA((NB,)),
            pltpu.SemaphoreType.DMA((NB,)),
        ],
    )
    def gather_kernel(idx_hbm, table_hbm, out_hbm, idx_v, rows_v, g_sems, w_sems):
        wid = lax.axis_index("s") * NC + lax.axis_index("c")
        base = wid * RW
        pltpu.sync_copy(idx_hbm.at[pl.ds(base, RW)], idx_v)

        def start_gather(j, slot):
            pltpu.async_copy(
                table_hbm.at[idx_v.at[j]], rows_v.at[slot], g_sems.at[slot]
            )

        def wait_gather(slot):
            pltpu.make_async_copy(
                table_hbm.at[idx_v.at[0]], rows_v.at[slot], g_sems.at[slot]
            ).wait()

        def wait_write(slot):
            pltpu.make_async_copy(
                rows_v.at[slot, :, pl.ds(0, D)], out_hbm.at[base], w_sems.at[slot]
            ).wait()

        for c in range(K):
            start_gather(c, c)

        def outer(o, carry):
            for b in range(NB):
                j = o * NB + b
                s = (b + K) % NB
                @pl.when(j + K < RW)
                def _():
                    @pl.when(j + K >= NB)
                    def _():
                        wait_write(s)
                    start_gather(j + K, s)
                wait_gather(b)
                pltpu.async_copy(
                    rows_v.at[b, :, pl.ds(0, D)], out_hbm.at[base + j], w_sems.at[b]
                )
            return carry

        lax.fori_loop(0, RW // NB, outer, 0)
        for b in range(NB):
            wait_write(b)

    return gather_kernel


def kernel(input_, weight):
    Bt, S = input_.shape
    D = weight.shape[1]
    wp = jnp.pad(weight, ((0, 0), (0, D)))
    return _make_gather(Bt, S, D)(input_, wp)
